# baseline (speedup 1.0000x reference)
import jax
import jax.numpy as jnp
import numpy as np
from jax import lax
from jax.experimental import pallas as pl
from jax.experimental.pallas import tpu as pltpu
from jax.experimental.pallas import tpu_sc as plsc

N = 10000
D = 128
E = 320000
ITERS = 10
ALPHA_F = np.float32(0.9)
RES_F = np.float32(1.0 - 0.9)

NC = 2
NS = 16
DH = D // NC

CH = 128
EPT_REAL = E // NS
EPT = 20480
CPT = EPT // CH
GRP = 8
NGRP = CPT // GRP
DUMMY = N
N_T = N + 16

NPT = N // NS
BLK = 125
NBLK = NPT // BLK
ZB = 25

def _rsqrt16(x):
    y = x * np.float32(0.0) + np.float32(24.0)
    for _ in range(12):
        y = np.float32(0.5) * (y + x / y)
    return np.float32(1.0) / y


def _body(nlo_hbm, nhi_hbm, rowc_hbm, colc_hbm, olo_hbm, ohi_hbm,
          s_hbm, t_sh, idx_r, idx_c, gbuf, tbuf, nbuf, zbuf, dis_v,
          sem_g, sem_s, sem_s2, sem_i, sem_n, sem_t, sem_v, sem_z):
    c = lax.axis_index("c")
    w = lax.axis_index("s")
    nbase = w * NPT
    cbase = w * CPT

    zeros16 = lax.broadcast(np.float32(0.0), (16,))
    ones16 = lax.broadcast(np.float32(1.0), (16,))

    def load_nodes(rbase):
        @pl.when(c == 0)
        def _():
            pltpu.sync_copy(nlo_hbm.at[pl.ds(rbase * DH, BLK * DH)], nbuf)

        @pl.when(c == 1)
        def _():
            pltpu.sync_copy(nhi_hbm.at[pl.ds(rbase * DH, BLK * DH)], nbuf)

    def store_out(rbase):
        @pl.when(c == 0)
        def _():
            pltpu.sync_copy(nbuf, olo_hbm.at[pl.ds(rbase * DH, BLK * DH)])

        @pl.when(c == 1)
        def _():
            pltpu.sync_copy(nbuf, ohi_hbm.at[pl.ds(rbase * DH, BLK * DH)])

    def zero_t(rbase):
        for zb in range(BLK // ZB):
            pltpu.sync_copy(zbuf, t_sh.at[pl.ds(rbase + zb * ZB, ZB)])

    @pl.loop(0, ZB)
    def _fill_z(r):
        for q in range(4):
            zbuf[r, pl.ds(q * 16, 16)] = zeros16

    @pl.loop(0, CH)
    def _fill_o(r):
        for q in range(4):
            gbuf[0, r, pl.ds(q * 16, 16)] = ones16

    for b in range(NBLK):
        zero_t(nbase + b * BLK)
    plsc.subcore_barrier()

    pltpu.async_copy(colc_hbm.at[pl.ds(cbase, GRP)], idx_c.at[0], sem_i)

    @pl.loop(0, NGRP)
    def _deg(g):
        sl = g % 2
        pltpu.make_async_copy(colc_hbm.at[pl.ds(cbase, GRP)],
                              idx_c.at[sl], sem_i).wait()

        @pl.when(g > 0)
        def _():
            for j in range(GRP):
                pltpu.make_async_copy(gbuf.at[0], t_sh.at[idx_c.at[sl, j]],
                                      sem_s).wait()

        @pl.when(g < NGRP - 1)
        def _():
            pltpu.async_copy(colc_hbm.at[pl.ds(cbase + (g + 1) * GRP, GRP)],
                             idx_c.at[1 - sl], sem_i)

        for j in range(GRP):
            pltpu.async_copy(gbuf.at[0], t_sh.at[idx_c.at[sl, j]], sem_s,
                             add=True)

    for j in range(GRP):
        pltpu.make_async_copy(gbuf.at[0], t_sh.at[idx_c.at[0, j]],
                              sem_s).wait()
    plsc.subcore_barrier()

    for b in range(NBLK):
        rbase = nbase + b * BLK
        pltpu.sync_copy(t_sh.at[pl.ds(rbase, BLK)], tbuf)
        load_nodes(rbase)

        @pl.loop(0, BLK)
        def _dis(r, b=b):
            deg = tbuf[r, pl.ds(0, 16)]
            dsafe = jnp.maximum(deg, np.float32(1.0))
            y = _rsqrt16(dsafe)
            dis = jnp.where(deg >= np.float32(0.5), y, np.float32(0.0))
            dis_v[pl.ds((b * BLK + r) * 16, 16)] = dis
            for q in range(4):
                tbuf[r, pl.ds(q * 16, 16)] = (
                    dis * nbuf[pl.ds(r * DH + q * 16, 16)])

        pltpu.sync_copy(tbuf, s_hbm.at[pl.ds(c * N + rbase, BLK)])
        zero_t(rbase)
    plsc.subcore_barrier()

    @pl.loop(0, ITERS)
    def _iter(it):
        pltpu.async_copy(rowc_hbm.at[c, pl.ds(cbase, GRP)], idx_r.at[0], sem_i)
        pltpu.async_copy(colc_hbm.at[pl.ds(cbase, GRP)], idx_c.at[0], sem_i)

        @pl.loop(0, NGRP)
        def _edges(g):
            sl = g % 2
            pltpu.make_async_copy(rowc_hbm.at[c, pl.ds(cbase, GRP)],
                                  idx_r.at[sl], sem_i).wait()
            pltpu.make_async_copy(colc_hbm.at[pl.ds(cbase, GRP)],
                                  idx_c.at[sl], sem_i).wait()

            @pl.when(g < NGRP - 1)
            def _():
                nb = cbase + (g + 1) * GRP
                pltpu.async_copy(rowc_hbm.at[c, pl.ds(nb, GRP)],
                                 idx_r.at[1 - sl], sem_i)
                pltpu.async_copy(colc_hbm.at[pl.ds(nb, GRP)],
                                 idx_c.at[1 - sl], sem_i)

            for j in range(GRP):
                p = j % 2
                sem_p = sem_s if p == 0 else sem_s2
                drain = pltpu.make_async_copy(
                    gbuf.at[p], t_sh.at[idx_c.at[sl, j]], sem_p)
                if j >= 2:
                    drain.wait()
                else:
                    @pl.when(g > 0)
                    def _(drain=drain):
                        drain.wait()
                pltpu.async_copy(s_hbm.at[idx_r.at[sl, j]], gbuf.at[p],
                                 sem_g).wait()
                pltpu.async_copy(gbuf.at[p], t_sh.at[idx_c.at[sl, j]], sem_p,
                                 add=True)
        pltpu.make_async_copy(gbuf.at[0], t_sh.at[idx_c.at[0, GRP - 2]],
                              sem_s).wait()
        pltpu.make_async_copy(gbuf.at[1], t_sh.at[idx_c.at[0, GRP - 1]],
                              sem_s2).wait()
        plsc.subcore_barrier()

        for b in range(NBLK):
            rbase = nbase + b * BLK

            @pl.when(c == 0)
            def _(rbase=rbase):
                pltpu.async_copy(nlo_hbm.at[pl.ds(rbase * DH, BLK * DH)],
                                 nbuf, sem_n)

            @pl.when(c == 1)
            def _(rbase=rbase):
                pltpu.async_copy(nhi_hbm.at[pl.ds(rbase * DH, BLK * DH)],
                                 nbuf, sem_n)

            if b > 0:
                pltpu.make_async_copy(tbuf, s_hbm.at[pl.ds(c * N + rbase - BLK, BLK)],
                                      sem_v).wait()
            pltpu.async_copy(t_sh.at[pl.ds(rbase, BLK)], tbuf, sem_t)
            pltpu.make_async_copy(nlo_hbm.at[pl.ds(rbase * DH, BLK * DH)],
                                  nbuf, sem_n).wait()
            pltpu.make_async_copy(t_sh.at[pl.ds(rbase, BLK)], tbuf,
                                  sem_t).wait()
            if b > 0:
                for zb in range(BLK // ZB):
                    pltpu.make_async_copy(
                        zbuf, t_sh.at[pl.ds(rbase - BLK + zb * ZB, ZB)],
                        sem_z).wait()
            for zb in range(BLK // ZB):
                pltpu.async_copy(zbuf, t_sh.at[pl.ds(rbase + zb * ZB, ZB)],
                                 sem_z)

            @pl.loop(0, BLK)
            def _nodes(r, b=b):
                dis = dis_v[pl.ds((b * BLK + r) * 16, 16)]
                for q in range(4):
                    tsl = pl.ds(q * 16, 16)
                    nsl = pl.ds(r * DH + q * 16, 16)
                    o = ALPHA_F * dis * tbuf[r, tsl] + RES_F * nbuf[nsl]
                    o = jnp.minimum(jnp.maximum(o, np.float32(0.0)),
                                    np.float32(1.0))
                    nbuf[nsl] = o
                    tbuf[r, tsl] = dis * o

            @pl.when(it == ITERS - 1)
            def _(rbase=rbase):
                store_out(rbase)

            pltpu.async_copy(tbuf, s_hbm.at[pl.ds(c * N + rbase, BLK)], sem_v)

        last = nbase + (NBLK - 1) * BLK
        pltpu.make_async_copy(tbuf, s_hbm.at[pl.ds(c * N + last, BLK)], sem_v).wait()
        for zb in range(BLK // ZB):
            pltpu.make_async_copy(zbuf, t_sh.at[pl.ds(last + zb * ZB, ZB)],
                                  sem_z).wait()
        plsc.subcore_barrier()


_prop = pl.kernel(
    _body,
    out_type=(jax.ShapeDtypeStruct((N * DH,), jnp.float32),
              jax.ShapeDtypeStruct((N * DH,), jnp.float32)),
    mesh=plsc.VectorSubcoreMesh(core_axis_name="c", subcore_axis_name="s",
                                num_cores=NC, num_subcores=NS),
    compiler_params=pltpu.CompilerParams(use_tc_tiling_on_sc=False),
    scratch_types=[
        pltpu.HBM((NC * N, DH), jnp.float32),
        pltpu.VMEM_SHARED((N_T, DH), jnp.float32),
        pltpu.VMEM((2, GRP, CH), jnp.int32),
        pltpu.VMEM((2, GRP, CH), jnp.int32),
        pltpu.VMEM((2, CH, DH), jnp.float32),
        pltpu.VMEM((BLK, DH), jnp.float32),
        pltpu.VMEM((BLK * DH,), jnp.float32),
        pltpu.VMEM((ZB, DH), jnp.float32),
        pltpu.VMEM((NPT * 16,), jnp.float32),
        pltpu.SemaphoreType.DMA,
        pltpu.SemaphoreType.DMA,
        pltpu.SemaphoreType.DMA,
        pltpu.SemaphoreType.DMA,
        pltpu.SemaphoreType.DMA,
        pltpu.SemaphoreType.DMA,
        pltpu.SemaphoreType.DMA,
        pltpu.SemaphoreType.DMA,
    ],
)


def kernel(nodes, edge_index):
    row = edge_index[0].reshape(NS, EPT_REAL)
    col = edge_index[1].reshape(NS, EPT_REAL)
    pad_r = jnp.zeros((NS, EPT - EPT_REAL), jnp.int32)
    pad_c = jnp.full((NS, EPT - EPT_REAL), DUMMY, jnp.int32)
    rowc = jnp.concatenate([row, pad_r], axis=1).reshape(-1, CH)
    colc = jnp.concatenate([col, pad_c], axis=1).reshape(-1, CH)
    rowc = jnp.stack([rowc, rowc + N])
    nlo = nodes[:, :DH].reshape(-1)
    nhi = nodes[:, DH:].reshape(-1)
    olo, ohi = _prop(nlo, nhi, rowc, colc)
    return jnp.concatenate(
        [olo.reshape(N, DH), ohi.reshape(N, DH)], axis=1)


# speedup vs baseline: 1.9306x; 1.9306x over previous
import jax
import jax.numpy as jnp
import numpy as np
from jax import lax
from jax.experimental import pallas as pl
from jax.experimental.pallas import tpu as pltpu
from jax.experimental.pallas import tpu_sc as plsc

N = 10000
D = 128
E = 320000
ITERS = 10
ALPHA_F = np.float32(0.9)
RES_F = np.float32(1.0 - 0.9)

NC = 2
NS = 16
DH = D // NC

CH = 128
EPT_REAL = E // NS
EPT = 20480
CPT = EPT // CH
GRP = 8
NGRP = CPT // GRP
DUMMY = N
N_T = N + 16
NSLOT = 4

NPT = N // NS
BLK = 125
NBLK = NPT // BLK
ZB = 25


def _rsqrt16(x):
    y = x * np.float32(0.0) + np.float32(24.0)
    for _ in range(12):
        y = np.float32(0.5) * (y + x / y)
    return np.float32(1.0) / y


def _body(nlo_hbm, nhi_hbm, rowc_hbm, colc_hbm, olo_hbm, ohi_hbm,
          s_sh, t_sh, idx_r, idx_c, big, zbuf, dis_v,
          sem_ga, sem_gb, sem_c0, sem_c1, sem_c2, sem_c3,
          sem_i, sem_n, sem_t, sem_v, sem_z):
    c = lax.axis_index("c")
    w = lax.axis_index("s")
    nbase = w * NPT
    cbase = w * CPT
    sem_c = (sem_c0, sem_c1, sem_c2, sem_c3)
    sem_gg = (sem_ga, sem_gb)

    zeros16 = lax.broadcast(np.float32(0.0), (16,))
    ones16 = lax.broadcast(np.float32(1.0), (16,))

    def zero_t(rbase):
        for zb in range(BLK // ZB):
            pltpu.sync_copy(zbuf, t_sh.at[pl.ds(rbase + zb * ZB, ZB)])

    @pl.loop(0, ZB)
    def _fill_z(r):
        for q in range(4):
            zbuf[r, pl.ds(q * 16, 16)] = zeros16

    @pl.loop(0, CH)
    def _fill_o(r):
        for q in range(4):
            big[0, r, pl.ds(q * 16, 16)] = ones16

    for b in range(NBLK):
        zero_t(nbase + b * BLK)
    plsc.subcore_barrier()

    pltpu.async_copy(colc_hbm.at[pl.ds(cbase, GRP)], idx_c.at[0], sem_i)

    @pl.loop(0, NGRP)
    def _deg(g):
        sl = g % 2
        pltpu.make_async_copy(colc_hbm.at[pl.ds(cbase, GRP)],
                              idx_c.at[sl], sem_i).wait()

        @pl.when(g > 0)
        def _():
            for j in range(GRP):
                pltpu.make_async_copy(big.at[0], t_sh.at[idx_c.at[sl, j]],
                                      sem_c0).wait()

        @pl.when(g < NGRP - 1)
        def _():
            pltpu.async_copy(colc_hbm.at[pl.ds(cbase + (g + 1) * GRP, GRP)],
                             idx_c.at[1 - sl], sem_i)

        for j in range(GRP):
            pltpu.async_copy(big.at[0], t_sh.at[idx_c.at[sl, j]], sem_c0,
                             add=True)

    for j in range(GRP):
        pltpu.make_async_copy(big.at[0], t_sh.at[idx_c.at[0, j]],
                              sem_c0).wait()
    plsc.subcore_barrier()

    for b in range(NBLK):
        rbase = nbase + b * BLK
        pltpu.sync_copy(t_sh.at[pl.ds(rbase, BLK)], big.at[0, pl.ds(0, BLK)])

        @pl.when(c == 0)
        def _(rbase=rbase):
            pltpu.sync_copy(nlo_hbm.at[pl.ds(rbase, BLK)],
                            big.at[1, pl.ds(0, BLK)])

        @pl.when(c == 1)
        def _(rbase=rbase):
            pltpu.sync_copy(nhi_hbm.at[pl.ds(rbase, BLK)],
                            big.at[1, pl.ds(0, BLK)])

        @pl.loop(0, BLK)
        def _dis(r, b=b):
            deg = big[0, r, pl.ds(0, 16)]
            dsafe = jnp.maximum(deg, np.float32(1.0))
            y = _rsqrt16(dsafe)
            dis = jnp.where(deg >= np.float32(0.5), y, np.float32(0.0))
            dis_v[pl.ds((b * BLK + r) * 16, 16)] = dis
            for q in range(4):
                sl16 = pl.ds(q * 16, 16)
                big[0, r, sl16] = dis * big[1, r, sl16]

        pltpu.sync_copy(big.at[0, pl.ds(0, BLK)], s_sh.at[pl.ds(rbase, BLK)])
        zero_t(rbase)
    plsc.subcore_barrier()

    @pl.loop(0, ITERS)
    def _iter(it):
        pltpu.async_copy(rowc_hbm.at[pl.ds(cbase, GRP)], idx_r.at[0], sem_i)
        pltpu.async_copy(colc_hbm.at[pl.ds(cbase, GRP)], idx_c.at[0], sem_i)

        @pl.loop(0, NGRP)
        def _edges(g):
            sl = g % 2
            pltpu.make_async_copy(rowc_hbm.at[pl.ds(cbase, GRP)],
                                  idx_r.at[sl], sem_i).wait()
            pltpu.make_async_copy(colc_hbm.at[pl.ds(cbase, GRP)],
                                  idx_c.at[sl], sem_i).wait()

            @pl.when(g > 0)
            def _():
                for p in range(NSLOT):
                    pltpu.make_async_copy(big.at[p],
                                          t_sh.at[idx_c.at[sl, p]],
                                          sem_c[p]).wait()

            @pl.when(g < NGRP - 1)
            def _():
                nb = cbase + (g + 1) * GRP
                pltpu.async_copy(rowc_hbm.at[pl.ds(nb, GRP)],
                                 idx_r.at[1 - sl], sem_i)
                pltpu.async_copy(colc_hbm.at[pl.ds(nb, GRP)],
                                 idx_c.at[1 - sl], sem_i)

            pltpu.async_copy(s_sh.at[idx_r.at[sl, 0]], big.at[0], sem_ga)
            for j in range(GRP):
                if j < GRP - 1:
                    pn = (j + 1) % NSLOT
                    if j + 1 >= NSLOT:
                        pltpu.make_async_copy(
                            big.at[pn], t_sh.at[idx_c.at[sl, pn]],
                            sem_c[pn]).wait()
                    pltpu.async_copy(s_sh.at[idx_r.at[sl, j + 1]],
                                     big.at[pn], sem_gg[(j + 1) % 2])
                pltpu.make_async_copy(s_sh.at[idx_r.at[sl, j]],
                                      big.at[j % NSLOT],
                                      sem_gg[j % 2]).wait()
                pltpu.async_copy(big.at[j % NSLOT], t_sh.at[idx_c.at[sl, j]],
                                 sem_c[j % NSLOT], add=True)

        for p in range(NSLOT):
            pltpu.make_async_copy(big.at[p], t_sh.at[idx_c.at[0, p]],
                                  sem_c[p]).wait()
        plsc.subcore_barrier()

        for b in range(NBLK):
            rbase = nbase + b * BLK

            @pl.when(c == 0)
            def _(rbase=rbase):
                pltpu.async_copy(nlo_hbm.at[pl.ds(rbase, BLK)],
                                 big.at[1, pl.ds(0, BLK)], sem_n)

            @pl.when(c == 1)
            def _(rbase=rbase):
                pltpu.async_copy(nhi_hbm.at[pl.ds(rbase, BLK)],
                                 big.at[1, pl.ds(0, BLK)], sem_n)

            if b > 0:
                pltpu.make_async_copy(big.at[0, pl.ds(0, BLK)],
                                      s_sh.at[pl.ds(rbase - BLK, BLK)],
                                      sem_v).wait()
            pltpu.async_copy(t_sh.at[pl.ds(rbase, BLK)],
                             big.at[0, pl.ds(0, BLK)], sem_t)
            pltpu.make_async_copy(nlo_hbm.at[pl.ds(rbase, BLK)],
                                  big.at[1, pl.ds(0, BLK)], sem_n).wait()
            pltpu.make_async_copy(t_sh.at[pl.ds(rbase, BLK)],
                                  big.at[0, pl.ds(0, BLK)], sem_t).wait()
            if b > 0:
                for zb in range(BLK // ZB):
                    pltpu.make_async_copy(
                        zbuf, t_sh.at[pl.ds(rbase - BLK + zb * ZB, ZB)],
                        sem_z).wait()
            for zb in range(BLK // ZB):
                pltpu.async_copy(zbuf, t_sh.at[pl.ds(rbase + zb * ZB, ZB)],
                                 sem_z)

            @pl.loop(0, BLK)
            def _nodes(r, b=b):
                dis = dis_v[pl.ds((b * BLK + r) * 16, 16)]
                for q in range(4):
                    sl16 = pl.ds(q * 16, 16)
                    o = (ALPHA_F * dis * big[0, r, sl16]
                         + RES_F * big[1, r, sl16])
                    o = jnp.minimum(jnp.maximum(o, np.float32(0.0)),
                                    np.float32(1.0))
                    big[1, r, sl16] = o
                    big[0, r, sl16] = dis * o

            @pl.when(it == ITERS - 1)
            def _(rbase=rbase):
                @pl.when(c == 0)
                def _():
                    pltpu.sync_copy(big.at[1, pl.ds(0, BLK)],
                                    olo_hbm.at[pl.ds(rbase, BLK)])

                @pl.when(c == 1)
                def _():
                    pltpu.sync_copy(big.at[1, pl.ds(0, BLK)],
                                    ohi_hbm.at[pl.ds(rbase, BLK)])

            pltpu.async_copy(big.at[0, pl.ds(0, BLK)],
                             s_sh.at[pl.ds(rbase, BLK)], sem_v)

        last = nbase + (NBLK - 1) * BLK
        pltpu.make_async_copy(big.at[0, pl.ds(0, BLK)],
                              s_sh.at[pl.ds(last, BLK)], sem_v).wait()
        for zb in range(BLK // ZB):
            pltpu.make_async_copy(zbuf, t_sh.at[pl.ds(last + zb * ZB, ZB)],
                                  sem_z).wait()
        plsc.subcore_barrier()


_prop = pl.kernel(
    _body,
    out_type=(jax.ShapeDtypeStruct((N, DH), jnp.float32),
              jax.ShapeDtypeStruct((N, DH), jnp.float32)),
    mesh=plsc.VectorSubcoreMesh(core_axis_name="c", subcore_axis_name="s",
                                num_cores=NC, num_subcores=NS),
    compiler_params=pltpu.CompilerParams(use_tc_tiling_on_sc=False),
    scratch_types=[
        pltpu.VMEM_SHARED((N, DH), jnp.float32),
        pltpu.VMEM_SHARED((N_T, DH), jnp.float32),
        pltpu.VMEM((2, GRP, CH), jnp.int32),
        pltpu.VMEM((2, GRP, CH), jnp.int32),
        pltpu.VMEM((NSLOT, CH, DH), jnp.float32),
        pltpu.VMEM((ZB, DH), jnp.float32),
        pltpu.VMEM((NPT * 16,), jnp.float32),
        pltpu.SemaphoreType.DMA,
        pltpu.SemaphoreType.DMA,
        pltpu.SemaphoreType.DMA,
        pltpu.SemaphoreType.DMA,
        pltpu.SemaphoreType.DMA,
        pltpu.SemaphoreType.DMA,
        pltpu.SemaphoreType.DMA,
        pltpu.SemaphoreType.DMA,
        pltpu.SemaphoreType.DMA,
        pltpu.SemaphoreType.DMA,
        pltpu.SemaphoreType.DMA,
    ],
)


def kernel(nodes, edge_index):
    row = edge_index[0].reshape(NS, EPT_REAL)
    col = edge_index[1].reshape(NS, EPT_REAL)
    pad_r = jnp.zeros((NS, EPT - EPT_REAL), jnp.int32)
    pad_c = jnp.full((NS, EPT - EPT_REAL), DUMMY, jnp.int32)
    rowc = jnp.concatenate([row, pad_r], axis=1).reshape(-1, CH)
    colc = jnp.concatenate([col, pad_c], axis=1).reshape(-1, CH)
    nlo = nodes[:, :DH]
    nhi = nodes[:, DH:]
    olo, ohi = _prop(nlo, nhi, rowc, colc)
    return jnp.concatenate([olo, ohi], axis=1)


# speedup vs baseline: 2.1952x; 1.1370x over previous
import jax
import jax.numpy as jnp
import numpy as np
from jax import lax
from jax.experimental import pallas as pl
from jax.experimental.pallas import tpu as pltpu
from jax.experimental.pallas import tpu_sc as plsc

N = 10000
D = 128
E = 320000
ITERS = 10
ALPHA_F = np.float32(0.9)
RES_F = np.float32(1.0 - 0.9)

NC = 2
NS = 16
DH = D // NC

CH = 128
EPT_REAL = E // NS
EPT = 20480
CPT = EPT // CH
GRP = 8
NGRP = CPT // GRP
NIB = 3
DUMMY = N
N_T = N + 16
NSLOT = 4

NPT = N // NS
BLK = 125
NBLK = NPT // BLK
ZB = 25


def _rsqrt16(x):
    y = x * np.float32(0.0) + np.float32(24.0)
    for _ in range(12):
        y = np.float32(0.5) * (y + x / y)
    return np.float32(1.0) / y


def _body(nlo_hbm, nhi_hbm, rowc_hbm, colc_hbm, olo_hbm, ohi_hbm,
          s_sh, t_sh, idx_r, idx_c, big, zbuf, dis_v,
          sem_ga, sem_c0, sem_c1, sem_i,
          sem_n0, sem_n1, sem_t0, sem_t1, sem_v, sem_z):
    c = lax.axis_index("c")
    w = lax.axis_index("s")
    nbase = w * NPT
    cbase = w * CPT
    sem_c = (sem_c0, sem_c1)
    sem_n = (sem_n0, sem_n1)
    sem_t = (sem_t0, sem_t1)

    zeros16 = lax.broadcast(np.float32(0.0), (16,))
    ones16 = lax.broadcast(np.float32(1.0), (16,))

    def zero_t(rbase):
        for zb in range(BLK // ZB):
            pltpu.sync_copy(zbuf, t_sh.at[pl.ds(rbase + zb * ZB, ZB)])

    def nodes_blk(b):
        return (nlo_hbm, nhi_hbm), pl.ds(nbase + b * BLK, BLK)

    @pl.loop(0, ZB)
    def _fill_z(r):
        for q in range(4):
            zbuf[r, pl.ds(q * 16, 16)] = zeros16

    @pl.loop(0, CH)
    def _fill_o(r):
        for q in range(4):
            big[0, r, pl.ds(q * 16, 16)] = ones16

    for b in range(NBLK):
        zero_t(nbase + b * BLK)
    plsc.subcore_barrier()

    pltpu.async_copy(colc_hbm.at[pl.ds(cbase, GRP)], idx_c.at[0], sem_i)

    @pl.loop(0, NGRP)
    def _deg(g):
        sl = lax.rem(g, NIB)
        pltpu.make_async_copy(colc_hbm.at[pl.ds(cbase, GRP)],
                              idx_c.at[sl], sem_i).wait()

        @pl.when(g > 0)
        def _():
            for j in range(GRP):
                pltpu.make_async_copy(big.at[0], t_sh.at[idx_c.at[sl, j]],
                                      sem_c0).wait()

        @pl.when(g < NGRP - 1)
        def _():
            pltpu.async_copy(colc_hbm.at[pl.ds(cbase + (g + 1) * GRP, GRP)],
                             idx_c.at[lax.rem(g + 1, NIB)], sem_i)

        for j in range(GRP):
            pltpu.async_copy(big.at[0], t_sh.at[idx_c.at[sl, j]], sem_c0,
                             add=True)

    for j in range(GRP):
        pltpu.make_async_copy(big.at[0], t_sh.at[idx_c.at[0, j]],
                              sem_c0).wait()
    plsc.subcore_barrier()

    for b in range(NBLK):
        rbase = nbase + b * BLK
        pltpu.sync_copy(t_sh.at[pl.ds(rbase, BLK)], big.at[0, pl.ds(0, BLK)])

        @pl.when(c == 0)
        def _(rbase=rbase):
            pltpu.sync_copy(nlo_hbm.at[pl.ds(rbase, BLK)],
                            big.at[1, pl.ds(0, BLK)])

        @pl.when(c == 1)
        def _(rbase=rbase):
            pltpu.sync_copy(nhi_hbm.at[pl.ds(rbase, BLK)],
                            big.at[1, pl.ds(0, BLK)])

        @pl.loop(0, BLK)
        def _dis(r, b=b):
            deg = big[0, r, pl.ds(0, 16)]
            dsafe = jnp.maximum(deg, np.float32(1.0))
            y = _rsqrt16(dsafe)
            dis = jnp.where(deg >= np.float32(0.5), y, np.float32(0.0))
            dis_v[pl.ds((b * BLK + r) * 16, 16)] = dis
            for q in range(4):
                sl16 = pl.ds(q * 16, 16)
                big[0, r, sl16] = dis * big[1, r, sl16]

        pltpu.sync_copy(big.at[0, pl.ds(0, BLK)], s_sh.at[pl.ds(rbase, BLK)])
        zero_t(rbase)
    plsc.subcore_barrier()

    @pl.loop(0, ITERS)
    def _iter(it):
        pltpu.async_copy(rowc_hbm.at[pl.ds(cbase, GRP)], idx_r.at[0], sem_i)
        pltpu.async_copy(colc_hbm.at[pl.ds(cbase, GRP)], idx_c.at[0], sem_i)

        @pl.loop(0, NGRP)
        def _edges(g):
            sl = lax.rem(g, NIB)
            pltpu.make_async_copy(rowc_hbm.at[pl.ds(cbase, GRP)],
                                  idx_r.at[sl], sem_i).wait()
            pltpu.make_async_copy(colc_hbm.at[pl.ds(cbase, GRP)],
                                  idx_c.at[sl], sem_i).wait()

            @pl.when(g < NGRP - 1)
            def _():
                nb = cbase + (g + 1) * GRP
                nsl = lax.rem(g + 1, NIB)
                pltpu.async_copy(rowc_hbm.at[pl.ds(nb, GRP)],
                                 idx_r.at[nsl], sem_i)
                pltpu.async_copy(colc_hbm.at[pl.ds(nb, GRP)],
                                 idx_c.at[nsl], sem_i)

            for j in range(GRP):
                p = j % 2
                drain = pltpu.make_async_copy(
                    big.at[p], t_sh.at[idx_c.at[sl, j]], sem_c[p])
                if j >= 2:
                    drain.wait()
                else:
                    @pl.when(g > 0)
                    def _(drain=drain):
                        drain.wait()
                pltpu.async_copy(s_sh.at[idx_r.at[sl, j]], big.at[p],
                                 sem_ga).wait()
                pltpu.async_copy(big.at[p], t_sh.at[idx_c.at[sl, j]],
                                 sem_c[p], add=True)
        pltpu.make_async_copy(big.at[0], t_sh.at[idx_c.at[0, GRP - 2]],
                              sem_c0).wait()
        pltpu.make_async_copy(big.at[1], t_sh.at[idx_c.at[0, GRP - 1]],
                              sem_c1).wait()
        plsc.subcore_barrier()

        def issue_loads(b):
            pb = b % 2
            rs = pl.ds(nbase + b * BLK, BLK)

            @pl.when(c == 0)
            def _():
                pltpu.async_copy(nlo_hbm.at[rs],
                                 big.at[2 * pb + 1, pl.ds(0, BLK)], sem_n[pb])

            @pl.when(c == 1)
            def _():
                pltpu.async_copy(nhi_hbm.at[rs],
                                 big.at[2 * pb + 1, pl.ds(0, BLK)], sem_n[pb])

            pltpu.async_copy(t_sh.at[rs], big.at[2 * pb, pl.ds(0, BLK)],
                             sem_t[pb])

        issue_loads(0)
        for b in range(NBLK):
            pb = b % 2
            rbase = nbase + b * BLK
            if b + 1 < NBLK:
                if b >= 1:
                    pltpu.make_async_copy(
                        big.at[2 * (1 - pb), pl.ds(0, BLK)],
                        s_sh.at[pl.ds(rbase - BLK, BLK)], sem_v).wait()
                issue_loads(b + 1)
            pltpu.make_async_copy(nlo_hbm.at[pl.ds(rbase, BLK)],
                                  big.at[2 * pb + 1, pl.ds(0, BLK)],
                                  sem_n[pb]).wait()
            pltpu.make_async_copy(t_sh.at[pl.ds(rbase, BLK)],
                                  big.at[2 * pb, pl.ds(0, BLK)],
                                  sem_t[pb]).wait()
            if b > 0:
                for zb in range(BLK // ZB):
                    pltpu.make_async_copy(
                        zbuf, t_sh.at[pl.ds(rbase - BLK + zb * ZB, ZB)],
                        sem_z).wait()
            for zb in range(BLK // ZB):
                pltpu.async_copy(zbuf, t_sh.at[pl.ds(rbase + zb * ZB, ZB)],
                                 sem_z)

            @pl.loop(0, BLK)
            def _nodes(r, b=b, pb=pb):
                dis = dis_v[pl.ds((b * BLK + r) * 16, 16)]
                for q in range(4):
                    sl16 = pl.ds(q * 16, 16)
                    o = (ALPHA_F * dis * big[2 * pb, r, sl16]
                         + RES_F * big[2 * pb + 1, r, sl16])
                    o = jnp.minimum(jnp.maximum(o, np.float32(0.0)),
                                    np.float32(1.0))
                    big[2 * pb + 1, r, sl16] = o
                    big[2 * pb, r, sl16] = dis * o

            @pl.when(it == ITERS - 1)
            def _(rbase=rbase, pb=pb):
                @pl.when(c == 0)
                def _():
                    pltpu.sync_copy(big.at[2 * pb + 1, pl.ds(0, BLK)],
                                    olo_hbm.at[pl.ds(rbase, BLK)])

                @pl.when(c == 1)
                def _():
                    pltpu.sync_copy(big.at[2 * pb + 1, pl.ds(0, BLK)],
                                    ohi_hbm.at[pl.ds(rbase, BLK)])

            pltpu.async_copy(big.at[2 * pb, pl.ds(0, BLK)],
                             s_sh.at[pl.ds(rbase, BLK)], sem_v)

        for b in (NBLK - 2, NBLK - 1):
            pltpu.make_async_copy(
                big.at[2 * (b % 2), pl.ds(0, BLK)],
                s_sh.at[pl.ds(nbase + b * BLK, BLK)], sem_v).wait()
        last = nbase + (NBLK - 1) * BLK
        for zb in range(BLK // ZB):
            pltpu.make_async_copy(zbuf, t_sh.at[pl.ds(last + zb * ZB, ZB)],
                                  sem_z).wait()
        plsc.subcore_barrier()


_prop = pl.kernel(
    _body,
    out_type=(jax.ShapeDtypeStruct((N, DH), jnp.float32),
              jax.ShapeDtypeStruct((N, DH), jnp.float32)),
    mesh=plsc.VectorSubcoreMesh(core_axis_name="c", subcore_axis_name="s",
                                num_cores=NC, num_subcores=NS),
    compiler_params=pltpu.CompilerParams(use_tc_tiling_on_sc=False),
    scratch_types=[
        pltpu.VMEM_SHARED((N, DH), jnp.float32),
        pltpu.VMEM_SHARED((N_T, DH), jnp.float32),
        pltpu.VMEM((NIB, GRP, CH), jnp.int32),
        pltpu.VMEM((NIB, GRP, CH), jnp.int32),
        pltpu.VMEM((NSLOT, CH, DH), jnp.float32),
        pltpu.VMEM((ZB, DH), jnp.float32),
        pltpu.VMEM((NPT * 16,), jnp.float32),
        pltpu.SemaphoreType.DMA,
        pltpu.SemaphoreType.DMA,
        pltpu.SemaphoreType.DMA,
        pltpu.SemaphoreType.DMA,
        pltpu.SemaphoreType.DMA,
        pltpu.SemaphoreType.DMA,
        pltpu.SemaphoreType.DMA,
        pltpu.SemaphoreType.DMA,
        pltpu.SemaphoreType.DMA,
        pltpu.SemaphoreType.DMA,
    ],
)


def kernel(nodes, edge_index):
    row = edge_index[0].reshape(NS, EPT_REAL)
    col = edge_index[1].reshape(NS, EPT_REAL)
    pad_r = jnp.zeros((NS, EPT - EPT_REAL), jnp.int32)
    pad_c = jnp.full((NS, EPT - EPT_REAL), DUMMY, jnp.int32)
    rowc = jnp.concatenate([row, pad_r], axis=1).reshape(-1, CH)
    colc = jnp.concatenate([col, pad_c], axis=1).reshape(-1, CH)
    nlo = nodes[:, :DH]
    nhi = nodes[:, DH:]
    olo, ohi = _prop(nlo, nhi, rowc, colc)
    return jnp.concatenate([olo, ohi], axis=1)
